# Initial kernel scaffold; baseline (speedup 1.0000x reference)
import jax
import jax.numpy as jnp
from jax.experimental import pallas as pl


def kernel(x, table):
    raise NotImplementedError("write your pallas kernel here")


# speedup vs baseline: 1.1766x; 1.1766x over previous
import functools

import jax
import jax.numpy as jnp
from jax import lax
from jax.experimental import pallas as pl
from jax.experimental.pallas import tpu as pltpu
from jax.experimental.pallas import tpu_sc as plsc

NC = 2
NS = 16
NW = NC * NS

CHUNK = 128
GROUP = 13


def _make_gather(total, v, d):
    per_w = total // NW
    n_chunk = per_w // CHUNK
    n_group = n_chunk // GROUP
    rows = GROUP * CHUNK
    assert per_w * NW == total and n_chunk * CHUNK == per_w
    assert n_group * GROUP == n_chunk

    mesh = plsc.VectorSubcoreMesh(core_axis_name="c", subcore_axis_name="s")

    @functools.partial(
        pl.kernel,
        mesh=mesh,
        compiler_params=pltpu.CompilerParams(use_tc_tiling_on_sc=False),
        out_type=jax.ShapeDtypeStruct((total, d), jnp.float32),
        scratch_types=[
            pltpu.VMEM((n_chunk, CHUNK), jnp.int32),
            pltpu.VMEM((rows, d), jnp.float32),
            pltpu.SemaphoreType.DMA,
        ],
    )
    def gather_kernel(idx_hbm, tab_hbm, out_hbm, idx_v, rows_v, gsem):
        wid = lax.axis_index("s") * NC + lax.axis_index("c")
        pltpu.sync_copy(idx_hbm.at[pl.ds(wid * n_chunk, n_chunk)], idx_v)

        def group_body(g, carry):
            handles = []
            for b in range(GROUP):
                h = pltpu.async_copy(
                    tab_hbm.at[idx_v.at[g * GROUP + b]],
                    rows_v.at[pl.ds(b * CHUNK, CHUNK)],
                    gsem,
                )
                handles.append(h)
            for h in handles:
                h.wait()
            pltpu.sync_copy(
                rows_v, out_hbm.at[pl.ds(wid * per_w + g * rows, rows)])
            return carry

        lax.fori_loop(0, n_group, group_body, 0)

    return gather_kernel


def kernel(x, table):
    b, f = x.shape
    v, d = table.shape
    total = b * f
    idx = x.reshape(total // CHUNK, CHUNK).astype(jnp.int32)
    out = _make_gather(total, v, d)(idx, table)
    return out.reshape(b, f * d)
